# baseline (speedup 1.0000x reference)
import functools

import jax
import jax.numpy as jnp
from jax import lax
from jax.experimental import pallas as pl
from jax.experimental.pallas import tpu as pltpu
from jax.experimental.pallas import tpu_sc as plsc

_N = 10000
_E = 320000
_D = 128
_NS = 16
_K = 80
_EPT = _E // (2 * _NS)
_CPT = _EPT // _K
_NSLAB = 5
_NCHUNK = _CPT // _NSLAB
_NP = 10240
_RPT = _NP // _NS
_RB = 2000

def _dot(a, b):
    return jnp.dot(a, b, preferred_element_type=jnp.float32)


def _tc_in_body(x_ref, win_ref, bin_ref, cw_ref, cb_ref, ew_ref, eb_ref,
                h_ref, e_ref):
    x1 = _dot(x_ref[...], win_ref[...]) + bin_ref[...]
    e_ref[...] = _dot(x1, ew_ref[...]) + eb_ref[...]
    h_ref[...] = _dot(x1, cw_ref[...]) + cb_ref[...]


def _tc_in(x, W_in, b_in, cW, cb, eWt, ebt):
    return pl.pallas_call(
        _tc_in_body,
        grid=(_N // _RB,),
        in_specs=[
            pl.BlockSpec((_RB, _D), lambda g: (g, 0)),
            pl.BlockSpec((_D, _D), lambda g: (0, 0)),
            pl.BlockSpec((1, _D), lambda g: (0, 0)),
            pl.BlockSpec((_D, _D), lambda g: (0, 0)),
            pl.BlockSpec((1, _D), lambda g: (0, 0)),
            pl.BlockSpec((_D, 1), lambda g: (0, 0)),
            pl.BlockSpec((1, 1), lambda g: (0, 0)),
        ],
        out_specs=[
            pl.BlockSpec((_RB, _D), lambda g: (g, 0)),
            pl.BlockSpec((_RB, 1), lambda g: (g, 0)),
        ],
        out_shape=[
            jax.ShapeDtypeStruct((_NP, _D), jnp.float32),
            jax.ShapeDtypeStruct((_N, 1), jnp.float32),
        ],
    )(x, W_in, b_in, cW, cb, eWt, ebt)


def _tc_mid_body(a0_ref, a1_ref, ep_ref, cw_ref, cb_ref, ew_ref, eb_ref,
                 h_ref, e_ref):
    xa = a0_ref[0] + a1_ref[0]
    x2 = jnp.where(xa > 0, xa, 0.01 * xa)
    e_ref[...] = ep_ref[...] + _dot(x2, ew_ref[...]) + eb_ref[...]
    h_ref[...] = _dot(x2, cw_ref[...]) + cb_ref[...]


def _tc_mid(agg, e_prev, cW, cb, eWt, ebt):
    return pl.pallas_call(
        _tc_mid_body,
        grid=(_N // _RB,),
        in_specs=[
            pl.BlockSpec((1, _RB, _D), lambda g: (0, g, 0)),
            pl.BlockSpec((1, _RB, _D), lambda g: (1, g, 0)),
            pl.BlockSpec((_RB, 1), lambda g: (g, 0)),
            pl.BlockSpec((_D, _D), lambda g: (0, 0)),
            pl.BlockSpec((1, _D), lambda g: (0, 0)),
            pl.BlockSpec((_D, 1), lambda g: (0, 0)),
            pl.BlockSpec((1, 1), lambda g: (0, 0)),
        ],
        out_specs=[
            pl.BlockSpec((_RB, _D), lambda g: (g, 0)),
            pl.BlockSpec((_RB, 1), lambda g: (g, 0)),
        ],
        out_shape=[
            jax.ShapeDtypeStruct((_NP, _D), jnp.float32),
            jax.ShapeDtypeStruct((_N, 1), jnp.float32),
        ],
    )(agg, agg, e_prev, cW, cb, eWt, ebt)


def _tc_out_body(a0_ref, a1_ref, ep_ref, ew_ref, eb_ref, e_ref):
    xa = a0_ref[0] + a1_ref[0]
    x3 = jnp.where(xa > 0, xa, 0.01 * xa)
    e_ref[...] = ep_ref[...] + _dot(x3, ew_ref[...]) + eb_ref[...]


def _tc_out(agg, e_prev, eWt, ebt):
    return pl.pallas_call(
        _tc_out_body,
        grid=(_N // _RB,),
        in_specs=[
            pl.BlockSpec((1, _RB, _D), lambda g: (0, g, 0)),
            pl.BlockSpec((1, _RB, _D), lambda g: (1, g, 0)),
            pl.BlockSpec((_RB, 1), lambda g: (g, 0)),
            pl.BlockSpec((_D, 1), lambda g: (0, 0)),
            pl.BlockSpec((1, 1), lambda g: (0, 0)),
        ],
        out_specs=pl.BlockSpec((_RB, 1), lambda g: (g, 0)),
        out_shape=jax.ShapeDtypeStruct((_N, 1), jnp.float32),
    )(agg, agg, e_prev, eWt, ebt)


@functools.partial(
    pl.kernel,
    out_type=jax.ShapeDtypeStruct((2, _NP, _D), jnp.float32),
    mesh=plsc.VectorSubcoreMesh(core_axis_name="c", subcore_axis_name="s"),
    scratch_types=[
        pltpu.VMEM_SHARED((_NP, _D), jnp.float32),
        pltpu.VMEM((_NCHUNK, _K), jnp.int32),
        pltpu.VMEM((_NCHUNK, _K), jnp.int32),
        pltpu.VMEM((_NCHUNK, _K), jnp.int32),
        pltpu.VMEM((_NCHUNK, _K), jnp.float32),
        pltpu.VMEM((_K, _D), jnp.float32),
        pltpu.VMEM((_K, _D), jnp.float32),
        pltpu.VMEM((_K, _D), jnp.float32),
        pltpu.SemaphoreType.DMA,
        pltpu.SemaphoreType.DMA,
        pltpu.SemaphoreType.DMA,
        pltpu.SemaphoreType.DMA,
        pltpu.SemaphoreType.DMA,
        pltpu.SemaphoreType.DMA,
    ],
)
def _sc_sweep(h_hbm, src_hbm, dst_hbm, w_hbm, out_hbm,
              agg_sh, srcA, srcB, dst_v, w_v, rows0, rows1, rows2,
              semg0, semg1, semg2, sems0, sems1, sems2):
    cid = lax.axis_index("c")
    sid = lax.axis_index("s")

    def _z(r, _):
        for c in range(_D // 16):
            rows0[r, pl.ds(c * 16, 16)] = jnp.zeros((16,), jnp.float32)
        return 0
    lax.fori_loop(0, _K, _z, 0)
    for j in range(_RPT // _K):
        pltpu.sync_copy(rows0, agg_sh.at[pl.ds(sid * _RPT + j * _K, _K)])
    plsc.subcore_barrier()

    pltpu.sync_copy(src_hbm.at[cid, sid, 0], srcA)
    pltpu.sync_copy(dst_hbm.at[cid, sid, 0], dst_v)
    pltpu.sync_copy(w_hbm.at[cid, sid, 0], w_v)
    pltpu.async_copy(h_hbm.at[srcA.at[0]], rows0, semg0)
    pltpu.async_copy(h_hbm.at[srcA.at[1]], rows1, semg1)

    bufs = ((rows0, semg0, sems0), (rows1, semg1, sems1),
            (rows2, semg2, sems2))

    def _chunk(g, b):
        rows, semg, sems = bufs[b]
        prows, _psemg, psems = bufs[(b + 2) % 3]
        lg = lax.rem(g, _NCHUNK)
        n2 = g + 2

        pltpu.make_async_copy(h_hbm.at[pl.ds(0, _K)], rows, semg).wait()

        @pl.when(jnp.logical_and(lg == 0, g > 0))
        def _():
            pltpu.make_async_copy(prows, agg_sh.at[pl.ds(0, _K)],
                                  psems).wait()
            s = lax.div(g, _NCHUNK)
            pltpu.sync_copy(dst_hbm.at[cid, sid, s], dst_v)
            pltpu.sync_copy(w_hbm.at[cid, sid, s], w_v)

        @pl.when(jnp.logical_and(lg == 0, g + _NCHUNK < _CPT))
        def _():
            s1 = lax.div(g, _NCHUNK) + 1
            @pl.when(lax.rem(s1, 2) == 0)
            def _():
                pltpu.sync_copy(src_hbm.at[cid, sid, s1], srcA)

            @pl.when(lax.rem(s1, 2) == 1)
            def _():
                pltpu.sync_copy(src_hbm.at[cid, sid, s1], srcB)

        def _scale(blk, _2):
            w16 = w_v[lg, pl.ds(blk * 16, 16)]
            for j in range(16):
                e = blk * 16 + j
                w = w16[j]
                for c in range(_D // 16):
                    sl = pl.ds(c * 16, 16)
                    rows[e, sl] = rows[e, sl] * w
            return 0
        lax.fori_loop(0, _K // 16, _scale, 0)

        @pl.when(jnp.logical_and(lg != 0, g > 0))
        def _():
            pltpu.make_async_copy(prows, agg_sh.at[pl.ds(0, _K)],
                                  psems).wait()

        @pl.when(n2 < _CPT)
        def _():
            l2 = lax.rem(n2, _NCHUNK)

            @pl.when(lax.rem(lax.div(n2, _NCHUNK), 2) == 0)
            def _():
                pltpu.async_copy(h_hbm.at[srcA.at[l2]], prows, _psemg)

            @pl.when(lax.rem(lax.div(n2, _NCHUNK), 2) == 1)
            def _():
                pltpu.async_copy(h_hbm.at[srcB.at[l2]], prows, _psemg)

        pltpu.async_copy(rows, agg_sh.at[dst_v.at[lg]], sems, add=True)

    def _triple(i, _):
        for p in (0, 1, 2):
            _chunk(3 * i + p, p)
        return 0
    lax.fori_loop(0, _CPT // 3, _triple, 0)
    _chunk(jnp.int32(_CPT - 2), 0)
    _chunk(jnp.int32(_CPT - 1), 1)
    pltpu.make_async_copy(rows1, agg_sh.at[pl.ds(0, _K)], sems1).wait()
    plsc.subcore_barrier()

    pltpu.sync_copy(agg_sh.at[pl.ds(sid * _RPT, _RPT)],
                    out_hbm.at[cid, pl.ds(sid * _RPT, _RPT)])


def kernel(x, edge_index, edge_weight, W_in, b_in, conv_W, conv_b,
           energy_W, energy_b, temp):
    def _slab5(a):
        return a.reshape(2, _NS, _NSLAB, _NCHUNK, _K)

    src2 = _slab5(edge_index[0])
    dst2 = _slab5(edge_index[1])
    w2 = _slab5(edge_weight)
    eWt = energy_W * temp[:, None, None]
    ebt = (energy_b * temp[:, None]).reshape(-1, 1, 1)
    b_in2 = b_in.reshape(1, _D)
    cb2 = conv_b.reshape(-1, 1, _D)

    h1, e0 = _tc_in(x, W_in, b_in2, conv_W[0], cb2[0], eWt[0], ebt[0])
    agg1 = _sc_sweep(h1, src2, dst2, w2)
    h2, e01 = _tc_mid(agg1, e0, conv_W[1], cb2[1], eWt[1], ebt[1])
    agg2 = _sc_sweep(h2, src2, dst2, w2)
    return _tc_out(agg2, e01, eWt[2], ebt[2])


# speedup vs baseline: 1.0006x; 1.0006x over previous
import functools

import jax
import jax.numpy as jnp
from jax import lax
from jax.experimental import pallas as pl
from jax.experimental.pallas import tpu as pltpu
from jax.experimental.pallas import tpu_sc as plsc

_N = 10000
_E = 320000
_D = 128
_NS = 16
_K = 80
_EPT = _E // (2 * _NS)
_CPT = _EPT // _K
_NSLAB = 5
_NCHUNK = _CPT // _NSLAB
_NP = 10240
_RPT = _NP // _NS
_RB = 2000

def _dot(a, b):
    return jnp.dot(a.astype(jnp.bfloat16), b.astype(jnp.bfloat16),
                   preferred_element_type=jnp.float32)


def _tc_in_body(x_ref, win_ref, bin_ref, cw_ref, cb_ref, ew_ref, eb_ref,
                h_ref, e_ref):
    x1 = _dot(x_ref[...], win_ref[...]) + bin_ref[...]
    e_ref[...] = _dot(x1, ew_ref[...]) + eb_ref[...]
    h_ref[...] = _dot(x1, cw_ref[...]) + cb_ref[...]


def _tc_in(x, W_in, b_in, cW, cb, eWt, ebt):
    return pl.pallas_call(
        _tc_in_body,
        grid=(_N // _RB,),
        in_specs=[
            pl.BlockSpec((_RB, _D), lambda g: (g, 0)),
            pl.BlockSpec((_D, _D), lambda g: (0, 0)),
            pl.BlockSpec((1, _D), lambda g: (0, 0)),
            pl.BlockSpec((_D, _D), lambda g: (0, 0)),
            pl.BlockSpec((1, _D), lambda g: (0, 0)),
            pl.BlockSpec((_D, 1), lambda g: (0, 0)),
            pl.BlockSpec((1, 1), lambda g: (0, 0)),
        ],
        out_specs=[
            pl.BlockSpec((_RB, _D), lambda g: (g, 0)),
            pl.BlockSpec((_RB, 1), lambda g: (g, 0)),
        ],
        out_shape=[
            jax.ShapeDtypeStruct((_NP, _D), jnp.float32),
            jax.ShapeDtypeStruct((_N, 1), jnp.float32),
        ],
    )(x, W_in, b_in, cW, cb, eWt, ebt)


def _tc_mid_body(a0_ref, a1_ref, ep_ref, cw_ref, cb_ref, ew_ref, eb_ref,
                 h_ref, e_ref):
    xa = a0_ref[0] + a1_ref[0]
    x2 = jnp.where(xa > 0, xa, 0.01 * xa)
    e_ref[...] = ep_ref[...] + _dot(x2, ew_ref[...]) + eb_ref[...]
    h_ref[...] = _dot(x2, cw_ref[...]) + cb_ref[...]


def _tc_mid(agg, e_prev, cW, cb, eWt, ebt):
    return pl.pallas_call(
        _tc_mid_body,
        grid=(_N // _RB,),
        in_specs=[
            pl.BlockSpec((1, _RB, _D), lambda g: (0, g, 0)),
            pl.BlockSpec((1, _RB, _D), lambda g: (1, g, 0)),
            pl.BlockSpec((_RB, 1), lambda g: (g, 0)),
            pl.BlockSpec((_D, _D), lambda g: (0, 0)),
            pl.BlockSpec((1, _D), lambda g: (0, 0)),
            pl.BlockSpec((_D, 1), lambda g: (0, 0)),
            pl.BlockSpec((1, 1), lambda g: (0, 0)),
        ],
        out_specs=[
            pl.BlockSpec((_RB, _D), lambda g: (g, 0)),
            pl.BlockSpec((_RB, 1), lambda g: (g, 0)),
        ],
        out_shape=[
            jax.ShapeDtypeStruct((_NP, _D), jnp.float32),
            jax.ShapeDtypeStruct((_N, 1), jnp.float32),
        ],
    )(agg, agg, e_prev, cW, cb, eWt, ebt)


def _tc_out_body(a0_ref, a1_ref, ep_ref, ew_ref, eb_ref, e_ref):
    xa = a0_ref[0] + a1_ref[0]
    x3 = jnp.where(xa > 0, xa, 0.01 * xa)
    e_ref[...] = ep_ref[...] + _dot(x3, ew_ref[...]) + eb_ref[...]


def _tc_out(agg, e_prev, eWt, ebt):
    return pl.pallas_call(
        _tc_out_body,
        grid=(_N // _RB,),
        in_specs=[
            pl.BlockSpec((1, _RB, _D), lambda g: (0, g, 0)),
            pl.BlockSpec((1, _RB, _D), lambda g: (1, g, 0)),
            pl.BlockSpec((_RB, 1), lambda g: (g, 0)),
            pl.BlockSpec((_D, 1), lambda g: (0, 0)),
            pl.BlockSpec((1, 1), lambda g: (0, 0)),
        ],
        out_specs=pl.BlockSpec((_RB, 1), lambda g: (g, 0)),
        out_shape=jax.ShapeDtypeStruct((_N, 1), jnp.float32),
    )(agg, agg, e_prev, eWt, ebt)


@functools.partial(
    pl.kernel,
    out_type=jax.ShapeDtypeStruct((2, _NP, _D), jnp.float32),
    mesh=plsc.VectorSubcoreMesh(core_axis_name="c", subcore_axis_name="s"),
    scratch_types=[
        pltpu.VMEM_SHARED((_NP, _D), jnp.float32),
        pltpu.VMEM((_NCHUNK, _K), jnp.int32),
        pltpu.VMEM((_NCHUNK, _K), jnp.int32),
        pltpu.VMEM((_NCHUNK, _K), jnp.int32),
        pltpu.VMEM((_NCHUNK, _K), jnp.float32),
        pltpu.VMEM((_K, _D), jnp.float32),
        pltpu.VMEM((_K, _D), jnp.float32),
        pltpu.VMEM((_K, _D), jnp.float32),
        pltpu.SemaphoreType.DMA,
        pltpu.SemaphoreType.DMA,
        pltpu.SemaphoreType.DMA,
        pltpu.SemaphoreType.DMA,
        pltpu.SemaphoreType.DMA,
        pltpu.SemaphoreType.DMA,
    ],
)
def _sc_sweep(h_hbm, src_hbm, dst_hbm, w_hbm, out_hbm,
              agg_sh, srcA, srcB, dst_v, w_v, rows0, rows1, rows2,
              semg0, semg1, semg2, sems0, sems1, sems2):
    cid = lax.axis_index("c")
    sid = lax.axis_index("s")

    pltpu.sync_copy(src_hbm.at[cid, sid, 0], srcA)
    pltpu.sync_copy(dst_hbm.at[cid, sid, 0], dst_v)
    pltpu.sync_copy(w_hbm.at[cid, sid, 0], w_v)
    pltpu.async_copy(h_hbm.at[srcA.at[0]], rows0, semg0)
    pltpu.async_copy(h_hbm.at[srcA.at[1]], rows1, semg1)

    def _z(r, _):
        for c in range(_D // 16):
            rows2[r, pl.ds(c * 16, 16)] = jnp.zeros((16,), jnp.float32)
        return 0
    lax.fori_loop(0, _K, _z, 0)
    for j in range(_RPT // _K):
        pltpu.sync_copy(rows2, agg_sh.at[pl.ds(sid * _RPT + j * _K, _K)])
    plsc.subcore_barrier()

    bufs = ((rows0, semg0, sems0), (rows1, semg1, sems1),
            (rows2, semg2, sems2))

    def _chunk(g, b):
        rows, semg, sems = bufs[b]
        prows, _psemg, psems = bufs[(b + 2) % 3]
        lg = lax.rem(g, _NCHUNK)
        n2 = g + 2

        pltpu.make_async_copy(h_hbm.at[pl.ds(0, _K)], rows, semg).wait()

        @pl.when(jnp.logical_and(lg == 0, g > 0))
        def _():
            pltpu.make_async_copy(prows, agg_sh.at[pl.ds(0, _K)],
                                  psems).wait()
            s = lax.div(g, _NCHUNK)
            pltpu.sync_copy(dst_hbm.at[cid, sid, s], dst_v)
            pltpu.sync_copy(w_hbm.at[cid, sid, s], w_v)

        @pl.when(jnp.logical_and(lg == 0, g + _NCHUNK < _CPT))
        def _():
            s1 = lax.div(g, _NCHUNK) + 1
            @pl.when(lax.rem(s1, 2) == 0)
            def _():
                pltpu.sync_copy(src_hbm.at[cid, sid, s1], srcA)

            @pl.when(lax.rem(s1, 2) == 1)
            def _():
                pltpu.sync_copy(src_hbm.at[cid, sid, s1], srcB)

        def _scale(blk, _2):
            w16 = w_v[lg, pl.ds(blk * 16, 16)]
            for j in range(16):
                e = blk * 16 + j
                w = w16[j]
                for c in range(_D // 16):
                    sl = pl.ds(c * 16, 16)
                    rows[e, sl] = rows[e, sl] * w
            return 0
        lax.fori_loop(0, _K // 16, _scale, 0)

        @pl.when(jnp.logical_and(lg != 0, g > 0))
        def _():
            pltpu.make_async_copy(prows, agg_sh.at[pl.ds(0, _K)],
                                  psems).wait()

        @pl.when(n2 < _CPT)
        def _():
            l2 = lax.rem(n2, _NCHUNK)

            @pl.when(lax.rem(lax.div(n2, _NCHUNK), 2) == 0)
            def _():
                pltpu.async_copy(h_hbm.at[srcA.at[l2]], prows, _psemg)

            @pl.when(lax.rem(lax.div(n2, _NCHUNK), 2) == 1)
            def _():
                pltpu.async_copy(h_hbm.at[srcB.at[l2]], prows, _psemg)

        pltpu.async_copy(rows, agg_sh.at[dst_v.at[lg]], sems, add=True)

    def _triple(i, _):
        for p in (0, 1, 2):
            _chunk(3 * i + p, p)
        return 0
    lax.fori_loop(0, _CPT // 3, _triple, 0)
    _chunk(jnp.int32(_CPT - 2), 0)
    _chunk(jnp.int32(_CPT - 1), 1)
    pltpu.make_async_copy(rows1, agg_sh.at[pl.ds(0, _K)], sems1).wait()
    plsc.subcore_barrier()

    pltpu.sync_copy(agg_sh.at[pl.ds(sid * _RPT, _RPT)],
                    out_hbm.at[cid, pl.ds(sid * _RPT, _RPT)])


def kernel(x, edge_index, edge_weight, W_in, b_in, conv_W, conv_b,
           energy_W, energy_b, temp):
    def _slab5(a):
        return a.reshape(2, _NS, _NSLAB, _NCHUNK, _K)

    src2 = _slab5(edge_index[0])
    dst2 = _slab5(edge_index[1])
    w2 = _slab5(edge_weight)
    eWt = energy_W * temp[:, None, None]
    ebt = (energy_b * temp[:, None]).reshape(-1, 1, 1)
    b_in2 = b_in.reshape(1, _D)
    cb2 = conv_b.reshape(-1, 1, _D)

    h1, e0 = _tc_in(x, W_in, b_in2, conv_W[0], cb2[0], eWt[0], ebt[0])
    agg1 = _sc_sweep(h1, src2, dst2, w2)
    h2, e01 = _tc_mid(agg1, e0, conv_W[1], cb2[1], eWt[1], ebt[1])
    agg2 = _sc_sweep(h2, src2, dst2, w2)
    return _tc_out(agg2, e01, eWt[2], ebt[2])


# speedup vs baseline: 1.0066x; 1.0060x over previous
import functools

import jax
import jax.numpy as jnp
from jax import lax
from jax.experimental import pallas as pl
from jax.experimental.pallas import tpu as pltpu
from jax.experimental.pallas import tpu_sc as plsc

_N = 10000
_E = 320000
_D = 128
_NS = 16
_K = 80
_EPT = _E // (2 * _NS)
_CPT = _EPT // _K
_NSLAB = 5
_NCHUNK = _CPT // _NSLAB
_NP = 10240
_RPT = _NP // _NS
_RB = 2000

def _dot(a, b):
    return jnp.dot(a, b, preferred_element_type=jnp.float32)


def _tc_in_body(x_ref, win_ref, bin_ref, cw_ref, cb_ref, ew_ref, eb_ref,
                h_ref, e_ref):
    x1 = _dot(x_ref[...], win_ref[...]) + bin_ref[...]
    e_ref[...] = _dot(x1, ew_ref[...]) + eb_ref[...]
    h_ref[...] = _dot(x1, cw_ref[...]) + cb_ref[...]


def _tc_in(x, W_in, b_in, cW, cb, eWt, ebt):
    return pl.pallas_call(
        _tc_in_body,
        grid=(_N // _RB,),
        in_specs=[
            pl.BlockSpec((_RB, _D), lambda g: (g, 0)),
            pl.BlockSpec((_D, _D), lambda g: (0, 0)),
            pl.BlockSpec((1, _D), lambda g: (0, 0)),
            pl.BlockSpec((_D, _D), lambda g: (0, 0)),
            pl.BlockSpec((1, _D), lambda g: (0, 0)),
            pl.BlockSpec((_D, 1), lambda g: (0, 0)),
            pl.BlockSpec((1, 1), lambda g: (0, 0)),
        ],
        out_specs=[
            pl.BlockSpec((_RB, _D), lambda g: (g, 0)),
            pl.BlockSpec((_RB, 1), lambda g: (g, 0)),
        ],
        out_shape=[
            jax.ShapeDtypeStruct((_NP, _D), jnp.float32),
            jax.ShapeDtypeStruct((_N, 1), jnp.float32),
        ],
    )(x, W_in, b_in, cW, cb, eWt, ebt)


def _tc_mid_body(a0_ref, a1_ref, ep_ref, cw_ref, cb_ref, ew_ref, eb_ref,
                 h_ref, e_ref):
    xa = a0_ref[0] + a1_ref[0]
    x2 = jnp.where(xa > 0, xa, 0.01 * xa)
    e_ref[...] = ep_ref[...] + _dot(x2, ew_ref[...]) + eb_ref[...]
    h_ref[...] = _dot(x2, cw_ref[...]) + cb_ref[...]


def _tc_mid(agg, e_prev, cW, cb, eWt, ebt):
    return pl.pallas_call(
        _tc_mid_body,
        grid=(_N // _RB,),
        in_specs=[
            pl.BlockSpec((1, _RB, _D), lambda g: (0, g, 0)),
            pl.BlockSpec((1, _RB, _D), lambda g: (1, g, 0)),
            pl.BlockSpec((_RB, 1), lambda g: (g, 0)),
            pl.BlockSpec((_D, _D), lambda g: (0, 0)),
            pl.BlockSpec((1, _D), lambda g: (0, 0)),
            pl.BlockSpec((_D, 1), lambda g: (0, 0)),
            pl.BlockSpec((1, 1), lambda g: (0, 0)),
        ],
        out_specs=[
            pl.BlockSpec((_RB, _D), lambda g: (g, 0)),
            pl.BlockSpec((_RB, 1), lambda g: (g, 0)),
        ],
        out_shape=[
            jax.ShapeDtypeStruct((_NP, _D), jnp.float32),
            jax.ShapeDtypeStruct((_N, 1), jnp.float32),
        ],
    )(agg, agg, e_prev, cW, cb, eWt, ebt)


def _tc_out_body(a0_ref, a1_ref, ep_ref, ew_ref, eb_ref, e_ref):
    xa = a0_ref[0] + a1_ref[0]
    x3 = jnp.where(xa > 0, xa, 0.01 * xa)
    e_ref[...] = ep_ref[...] + _dot(x3, ew_ref[...]) + eb_ref[...]


def _tc_out(agg, e_prev, eWt, ebt):
    return pl.pallas_call(
        _tc_out_body,
        grid=(_N // _RB,),
        in_specs=[
            pl.BlockSpec((1, _RB, _D), lambda g: (0, g, 0)),
            pl.BlockSpec((1, _RB, _D), lambda g: (1, g, 0)),
            pl.BlockSpec((_RB, 1), lambda g: (g, 0)),
            pl.BlockSpec((_D, 1), lambda g: (0, 0)),
            pl.BlockSpec((1, 1), lambda g: (0, 0)),
        ],
        out_specs=pl.BlockSpec((_RB, 1), lambda g: (g, 0)),
        out_shape=jax.ShapeDtypeStruct((_N, 1), jnp.float32),
    )(agg, agg, e_prev, eWt, ebt)


@functools.partial(
    pl.kernel,
    out_type=jax.ShapeDtypeStruct((2, _NP, _D), jnp.float32),
    mesh=plsc.VectorSubcoreMesh(core_axis_name="c", subcore_axis_name="s"),
    scratch_types=[
        pltpu.VMEM_SHARED((_NP, _D), jnp.float32),
        pltpu.VMEM((_NCHUNK, _K), jnp.int32),
        pltpu.VMEM((_NCHUNK, _K), jnp.int32),
        pltpu.VMEM((_NCHUNK, _K), jnp.int32),
        pltpu.VMEM((_NCHUNK, _K), jnp.float32),
        pltpu.VMEM((_K, _D), jnp.float32),
        pltpu.VMEM((_K, _D), jnp.float32),
        pltpu.VMEM((_K, _D), jnp.float32),
        pltpu.SemaphoreType.DMA,
        pltpu.SemaphoreType.DMA,
        pltpu.SemaphoreType.DMA,
        pltpu.SemaphoreType.DMA,
        pltpu.SemaphoreType.DMA,
        pltpu.SemaphoreType.DMA,
    ],
)
def _sc_sweep(h_hbm, src_hbm, dst_hbm, w_hbm, out_hbm,
              agg_sh, srcA, srcB, dst_v, w_v, rows0, rows1, rows2,
              semg0, semg1, semg2, sems0, sems1, sems2):
    cid = lax.axis_index("c")
    sid = lax.axis_index("s")

    pltpu.sync_copy(src_hbm.at[cid, sid, 0], srcA)
    pltpu.sync_copy(dst_hbm.at[cid, sid, 0], dst_v)
    pltpu.sync_copy(w_hbm.at[cid, sid, 0], w_v)
    pltpu.async_copy(h_hbm.at[srcA.at[0]], rows0, semg0)
    pltpu.async_copy(h_hbm.at[srcA.at[1]], rows1, semg1)

    def _z(r, _):
        for c in range(_D // 16):
            rows2[r, pl.ds(c * 16, 16)] = jnp.zeros((16,), jnp.float32)
        return 0
    lax.fori_loop(0, _K, _z, 0)
    for j in range(_RPT // _K):
        pltpu.sync_copy(rows2, agg_sh.at[pl.ds(sid * _RPT + j * _K, _K)])
    plsc.subcore_barrier()

    bufs = ((rows0, semg0, sems0), (rows1, semg1, sems1),
            (rows2, semg2, sems2))

    def _chunk(g, b):
        rows, semg, sems = bufs[b]
        prows, _psemg, psems = bufs[(b + 2) % 3]
        lg = lax.rem(g, _NCHUNK)
        n2 = g + 2

        pltpu.make_async_copy(h_hbm.at[pl.ds(0, _K)], rows, semg).wait()

        @pl.when(jnp.logical_and(lg == 0, g > 0))
        def _():
            pltpu.make_async_copy(prows, agg_sh.at[pl.ds(0, _K)],
                                  psems).wait()
            s = lax.div(g, _NCHUNK)
            pltpu.sync_copy(dst_hbm.at[cid, sid, s], dst_v)
            pltpu.sync_copy(w_hbm.at[cid, sid, s], w_v)

        @pl.when(jnp.logical_and(lg == 0, g + _NCHUNK < _CPT))
        def _():
            s1 = lax.div(g, _NCHUNK) + 1
            @pl.when(lax.rem(s1, 2) == 0)
            def _():
                pltpu.sync_copy(src_hbm.at[cid, sid, s1], srcA)

            @pl.when(lax.rem(s1, 2) == 1)
            def _():
                pltpu.sync_copy(src_hbm.at[cid, sid, s1], srcB)

        def _scale(blk, _2):
            w16 = w_v[lg, pl.ds(blk * 16, 16)]
            for j in range(16):
                e = blk * 16 + j
                w = w16[j]
                for c in range(_D // 16):
                    sl = pl.ds(c * 16, 16)
                    rows[e, sl] = rows[e, sl] * w
            return 0
        lax.fori_loop(0, _K // 16, _scale, 0)

        @pl.when(jnp.logical_and(lg != 0, g > 0))
        def _():
            pltpu.make_async_copy(prows, agg_sh.at[pl.ds(0, _K)],
                                  psems).wait()

        @pl.when(n2 < _CPT)
        def _():
            l2 = lax.rem(n2, _NCHUNK)

            @pl.when(lax.rem(lax.div(n2, _NCHUNK), 2) == 0)
            def _():
                pltpu.async_copy(h_hbm.at[srcA.at[l2]], prows, _psemg)

            @pl.when(lax.rem(lax.div(n2, _NCHUNK), 2) == 1)
            def _():
                pltpu.async_copy(h_hbm.at[srcB.at[l2]], prows, _psemg)

        pltpu.async_copy(rows, agg_sh.at[dst_v.at[lg]], sems, add=True)

    def _triple(i, _):
        for p in (0, 1, 2):
            _chunk(3 * i + p, p)
        return 0
    lax.fori_loop(0, _CPT // 3, _triple, 0)
    _chunk(jnp.int32(_CPT - 2), 0)
    _chunk(jnp.int32(_CPT - 1), 1)
    pltpu.make_async_copy(rows1, agg_sh.at[pl.ds(0, _K)], sems1).wait()
    plsc.subcore_barrier()

    pltpu.sync_copy(agg_sh.at[pl.ds(sid * _RPT, _RPT)],
                    out_hbm.at[cid, pl.ds(sid * _RPT, _RPT)])


def kernel(x, edge_index, edge_weight, W_in, b_in, conv_W, conv_b,
           energy_W, energy_b, temp):
    def _slab5(a):
        return a.reshape(2, _NS, _NSLAB, _NCHUNK, _K)

    src2 = _slab5(edge_index[0])
    dst2 = _slab5(edge_index[1])
    w2 = _slab5(edge_weight)
    eWt = energy_W * temp[:, None, None]
    ebt = (energy_b * temp[:, None]).reshape(-1, 1, 1)
    b_in2 = b_in.reshape(1, _D)
    cb2 = conv_b.reshape(-1, 1, _D)

    h1, e0 = _tc_in(x, W_in, b_in2, conv_W[0], cb2[0], eWt[0], ebt[0])
    agg1 = _sc_sweep(h1, src2, dst2, w2)
    h2, e01 = _tc_mid(agg1, e0, conv_W[1], cb2[1], eWt[1], ebt[1])
    agg2 = _sc_sweep(h2, src2, dst2, w2)
    return _tc_out(agg2, e01, eWt[2], ebt[2])


# speedup vs baseline: 1.0066x; 1.0000x over previous
import functools

import jax
import jax.numpy as jnp
from jax import lax
from jax.experimental import pallas as pl
from jax.experimental.pallas import tpu as pltpu
from jax.experimental.pallas import tpu_sc as plsc

_N = 10000
_E = 320000
_D = 128
_NS = 16
_K = 80
_EPT = _E // (2 * _NS)
_CPT = _EPT // _K
_NSLAB = 5
_NCHUNK = _CPT // _NSLAB
_NP = 10240
_RPT = _NP // _NS
_RB = 2000

def _dot(a, b):
    return jnp.dot(a, b, preferred_element_type=jnp.float32)


def _tc_in_body(x_ref, win_ref, bin_ref, cw_ref, cb_ref, ew_ref, eb_ref,
                t_ref, h_ref, e_ref):
    x1 = _dot(x_ref[...], win_ref[...]) + bin_ref[...]
    e_ref[...] = (_dot(x1, ew_ref[...]) + eb_ref[...]) * t_ref[...]
    h_ref[...] = _dot(x1, cw_ref[...]) + cb_ref[...]


def _tc_in(x, W_in, b_in, cW, cb, eW, eb, t):
    return pl.pallas_call(
        _tc_in_body,
        grid=(_N // _RB,),
        in_specs=[
            pl.BlockSpec((_RB, _D), lambda g: (g, 0)),
            pl.BlockSpec((_D, _D), lambda g: (0, 0)),
            pl.BlockSpec((1, _D), lambda g: (0, 0)),
            pl.BlockSpec((_D, _D), lambda g: (0, 0)),
            pl.BlockSpec((1, _D), lambda g: (0, 0)),
            pl.BlockSpec((_D, 1), lambda g: (0, 0)),
            pl.BlockSpec((1, 1), lambda g: (0, 0)),
            pl.BlockSpec((1, 1), lambda g: (0, 0)),
        ],
        out_specs=[
            pl.BlockSpec((_RB, _D), lambda g: (g, 0)),
            pl.BlockSpec((_RB, 1), lambda g: (g, 0)),
        ],
        out_shape=[
            jax.ShapeDtypeStruct((_NP, _D), jnp.float32),
            jax.ShapeDtypeStruct((_N, 1), jnp.float32),
        ],
    )(x, W_in, b_in, cW, cb, eW, eb, t)


def _tc_mid_body(a0_ref, a1_ref, ep_ref, cw_ref, cb_ref, ew_ref, eb_ref,
                 t_ref, h_ref, e_ref):
    xa = a0_ref[0] + a1_ref[0]
    x2 = jnp.where(xa > 0, xa, 0.01 * xa)
    e_ref[...] = ep_ref[...] + (_dot(x2, ew_ref[...])
                                + eb_ref[...]) * t_ref[...]
    h_ref[...] = _dot(x2, cw_ref[...]) + cb_ref[...]


def _tc_mid(agg, e_prev, cW, cb, eW, eb, t):
    return pl.pallas_call(
        _tc_mid_body,
        grid=(_N // _RB,),
        in_specs=[
            pl.BlockSpec((1, _RB, _D), lambda g: (0, g, 0)),
            pl.BlockSpec((1, _RB, _D), lambda g: (1, g, 0)),
            pl.BlockSpec((_RB, 1), lambda g: (g, 0)),
            pl.BlockSpec((_D, _D), lambda g: (0, 0)),
            pl.BlockSpec((1, _D), lambda g: (0, 0)),
            pl.BlockSpec((_D, 1), lambda g: (0, 0)),
            pl.BlockSpec((1, 1), lambda g: (0, 0)),
            pl.BlockSpec((1, 1), lambda g: (0, 0)),
        ],
        out_specs=[
            pl.BlockSpec((_RB, _D), lambda g: (g, 0)),
            pl.BlockSpec((_RB, 1), lambda g: (g, 0)),
        ],
        out_shape=[
            jax.ShapeDtypeStruct((_NP, _D), jnp.float32),
            jax.ShapeDtypeStruct((_N, 1), jnp.float32),
        ],
    )(agg, agg, e_prev, cW, cb, eW, eb, t)


def _tc_out_body(a0_ref, a1_ref, ep_ref, ew_ref, eb_ref, t_ref, e_ref):
    xa = a0_ref[0] + a1_ref[0]
    x3 = jnp.where(xa > 0, xa, 0.01 * xa)
    e_ref[...] = ep_ref[...] + (_dot(x3, ew_ref[...])
                                + eb_ref[...]) * t_ref[...]


def _tc_out(agg, e_prev, eW, eb, t):
    return pl.pallas_call(
        _tc_out_body,
        grid=(_N // _RB,),
        in_specs=[
            pl.BlockSpec((1, _RB, _D), lambda g: (0, g, 0)),
            pl.BlockSpec((1, _RB, _D), lambda g: (1, g, 0)),
            pl.BlockSpec((_RB, 1), lambda g: (g, 0)),
            pl.BlockSpec((_D, 1), lambda g: (0, 0)),
            pl.BlockSpec((1, 1), lambda g: (0, 0)),
            pl.BlockSpec((1, 1), lambda g: (0, 0)),
        ],
        out_specs=pl.BlockSpec((_RB, 1), lambda g: (g, 0)),
        out_shape=jax.ShapeDtypeStruct((_N, 1), jnp.float32),
    )(agg, agg, e_prev, eW, eb, t)


@functools.partial(
    pl.kernel,
    out_type=jax.ShapeDtypeStruct((2, _NP, _D), jnp.float32),
    mesh=plsc.VectorSubcoreMesh(core_axis_name="c", subcore_axis_name="s"),
    scratch_types=[
        pltpu.VMEM_SHARED((_NP, _D), jnp.float32),
        pltpu.VMEM((_NCHUNK, _K), jnp.int32),
        pltpu.VMEM((_NCHUNK, _K), jnp.int32),
        pltpu.VMEM((_NCHUNK, _K), jnp.int32),
        pltpu.VMEM((_NCHUNK, _K), jnp.float32),
        pltpu.VMEM((_K, _D), jnp.float32),
        pltpu.VMEM((_K, _D), jnp.float32),
        pltpu.VMEM((_K, _D), jnp.float32),
        pltpu.SemaphoreType.DMA,
        pltpu.SemaphoreType.DMA,
        pltpu.SemaphoreType.DMA,
        pltpu.SemaphoreType.DMA,
        pltpu.SemaphoreType.DMA,
        pltpu.SemaphoreType.DMA,
    ],
)
def _sc_sweep(h_hbm, src_hbm, dst_hbm, w_hbm, out_hbm,
              agg_sh, srcA, srcB, dst_v, w_v, rows0, rows1, rows2,
              semg0, semg1, semg2, sems0, sems1, sems2):
    cid = lax.axis_index("c")
    sid = lax.axis_index("s")

    pltpu.sync_copy(src_hbm.at[cid, sid, 0], srcA)
    pltpu.sync_copy(dst_hbm.at[cid, sid, 0], dst_v)
    pltpu.sync_copy(w_hbm.at[cid, sid, 0], w_v)
    pltpu.async_copy(h_hbm.at[srcA.at[0]], rows0, semg0)
    pltpu.async_copy(h_hbm.at[srcA.at[1]], rows1, semg1)

    def _z(r, _):
        for c in range(_D // 16):
            rows2[r, pl.ds(c * 16, 16)] = jnp.zeros((16,), jnp.float32)
        return 0
    lax.fori_loop(0, _K, _z, 0)
    for j in range(_RPT // _K):
        pltpu.sync_copy(rows2, agg_sh.at[pl.ds(sid * _RPT + j * _K, _K)])
    plsc.subcore_barrier()

    bufs = ((rows0, semg0, sems0), (rows1, semg1, sems1),
            (rows2, semg2, sems2))

    def _chunk(g, b):
        rows, semg, sems = bufs[b]
        prows, _psemg, psems = bufs[(b + 2) % 3]
        lg = lax.rem(g, _NCHUNK)
        n2 = g + 2

        pltpu.make_async_copy(h_hbm.at[pl.ds(0, _K)], rows, semg).wait()

        @pl.when(jnp.logical_and(lg == 0, g > 0))
        def _():
            pltpu.make_async_copy(prows, agg_sh.at[pl.ds(0, _K)],
                                  psems).wait()
            s = lax.div(g, _NCHUNK)
            pltpu.sync_copy(dst_hbm.at[cid, sid, s], dst_v)
            pltpu.sync_copy(w_hbm.at[cid, sid, s], w_v)

        @pl.when(jnp.logical_and(lg == 0, g + _NCHUNK < _CPT))
        def _():
            s1 = lax.div(g, _NCHUNK) + 1
            @pl.when(lax.rem(s1, 2) == 0)
            def _():
                pltpu.sync_copy(src_hbm.at[cid, sid, s1], srcA)

            @pl.when(lax.rem(s1, 2) == 1)
            def _():
                pltpu.sync_copy(src_hbm.at[cid, sid, s1], srcB)

        def _scale(blk, _2):
            w16 = w_v[lg, pl.ds(blk * 16, 16)]
            for j in range(16):
                e = blk * 16 + j
                w = w16[j]
                for c in range(_D // 16):
                    sl = pl.ds(c * 16, 16)
                    rows[e, sl] = rows[e, sl] * w
            return 0
        lax.fori_loop(0, _K // 16, _scale, 0)

        @pl.when(jnp.logical_and(lg != 0, g > 0))
        def _():
            pltpu.make_async_copy(prows, agg_sh.at[pl.ds(0, _K)],
                                  psems).wait()

        @pl.when(n2 < _CPT)
        def _():
            l2 = lax.rem(n2, _NCHUNK)

            @pl.when(lax.rem(lax.div(n2, _NCHUNK), 2) == 0)
            def _():
                pltpu.async_copy(h_hbm.at[srcA.at[l2]], prows, _psemg)

            @pl.when(lax.rem(lax.div(n2, _NCHUNK), 2) == 1)
            def _():
                pltpu.async_copy(h_hbm.at[srcB.at[l2]], prows, _psemg)

        pltpu.async_copy(rows, agg_sh.at[dst_v.at[lg]], sems, add=True)

    def _triple(i, _):
        for p in (0, 1, 2):
            _chunk(3 * i + p, p)
        return 0
    lax.fori_loop(0, _CPT // 3, _triple, 0)
    _chunk(jnp.int32(_CPT - 2), 0)
    _chunk(jnp.int32(_CPT - 1), 1)
    pltpu.make_async_copy(rows1, agg_sh.at[pl.ds(0, _K)], sems1).wait()
    plsc.subcore_barrier()

    pltpu.sync_copy(agg_sh.at[pl.ds(sid * _RPT, _RPT)],
                    out_hbm.at[cid, pl.ds(sid * _RPT, _RPT)])


def kernel(x, edge_index, edge_weight, W_in, b_in, conv_W, conv_b,
           energy_W, energy_b, temp):
    def _slab5(a):
        return a.reshape(2, _NS, _NSLAB, _NCHUNK, _K)

    src2 = _slab5(edge_index[0])
    dst2 = _slab5(edge_index[1])
    w2 = _slab5(edge_weight)
    eb2 = energy_b.reshape(-1, 1, 1)
    t2 = temp.reshape(-1, 1, 1)
    b_in2 = b_in.reshape(1, _D)
    cb2 = conv_b.reshape(-1, 1, _D)

    h1, e0 = _tc_in(x, W_in, b_in2, conv_W[0], cb2[0],
                    energy_W[0], eb2[0], t2[0])
    agg1 = _sc_sweep(h1, src2, dst2, w2)
    h2, e01 = _tc_mid(agg1, e0, conv_W[1], cb2[1],
                      energy_W[1], eb2[1], t2[1])
    agg2 = _sc_sweep(h2, src2, dst2, w2)
    return _tc_out(agg2, e01, energy_W[2], eb2[2], t2[2])
